# baseline (speedup 1.0000x reference)
import functools

import jax
import jax.numpy as jnp
from jax import lax
from jax.experimental import pallas as pl
from jax.experimental.pallas import tpu as pltpu
from jax.experimental.pallas import tpu_sc as plsc

M, D, B = 100000, 256, 16384
EMA = 0.999
L = 16
NK = D // L
NC, NS = 2, 16
NW = NC * NS
RPW = B // NW
CH = 64
NCH = RPW // CH
OWN = 6256
PSZ = NS * OWN
NV = B // L
GW = 128

def _vrsqrt(s):
    s = jnp.maximum(s, 1e-24)
    i = plsc.bitcast(s, jnp.int32)
    y = plsc.bitcast(jnp.int32(0x5F3759DF) - (i >> 1), jnp.float32)
    for _ in range(3):
        y = y * (1.5 - 0.5 * s * y * y)
    return y


def _body(memh, confh, acth, idxh, valh, nch, outh, posh,
          idx_v, pos_v, gidx_v, lp_v, oc_v, aa_v, ncv_v, m_v,
          mrows0, mrows1, vrows0, vrows1, orows0, orows1,
          sem_s, sem_lp, sem_m0, sem_m1, sem_v0, sem_v1, sem_o0, sem_o1):
    c = lax.axis_index("c")
    s = lax.axis_index("s")
    wid = s * NC + c
    lane = lax.iota(jnp.int32, L)

    mrows = (mrows0, mrows1)
    vrows = (vrows0, vrows1)
    orows = (orows0, orows1)
    sem_m = (sem_m0, sem_m1)
    sem_v = (sem_v0, sem_v1)
    sem_o = (sem_o0, sem_o1)

    pltpu.sync_copy(idxh, idx_v)

    base = wid * RPW
    coff = c * PSZ

    early = []
    for g in range(RPW // GW):
        early.append(pltpu.async_copy(
            confh.at[idx_v.at[pl.ds(base + g * GW, GW)]],
            oc_v.at[pl.ds(g * GW, GW)], sem_s))
        early.append(pltpu.async_copy(
            acth.at[idx_v.at[pl.ds(base + g * GW, GW)]],
            aa_v.at[pl.ds(g * GW, GW)], sem_s))
    mem_cp = [pltpu.async_copy(memh.at[idx_v.at[pl.ds(base, CH)]],
                               mrows[0], sem_m[0])]

    for k in range(RPW // L):
        gidx_v[pl.ds(k * L, L)] = idx_v[pl.ds(base + k * L, L)] + coff

    lo = s * OWN

    def pos_step(v, carry):
        iv = idx_v[pl.ds(v * L, L)]
        loc = iv - lo
        own = plsc.bitcast(loc, jnp.uint32) < jnp.uint32(OWN)
        sj = v * L + lane
        plsc.store_scatter(pos_v, [loc], sj, mask=own)
        rb = plsc.load_gather(pos_v, [loc], mask=own)
        bad = jnp.logical_and(own, sj > rb)
        cnt = plsc.all_reduce_population_count(bad)

        @pl.when(cnt[0] > 0)
        def _fix():
            def fix_round(r, mcarry):
                plsc.store_scatter(pos_v, [loc], sj, mask=mcarry)
                rb2 = plsc.load_gather(pos_v, [loc], mask=own)
                return jnp.logical_and(own, sj > rb2)
            lax.fori_loop(0, L - 1, fix_round, bad)

        return carry

    lax.fori_loop(0, NV, pos_step, 0, unroll=4)

    pltpu.sync_copy(pos_v, posh.at[pl.ds(c * PSZ + lo, OWN)])
    plsc.subcore_barrier()

    cps = []
    for g in range(RPW // GW):
        cps.append(pltpu.async_copy(posh.at[gidx_v.at[pl.ds(g * GW, GW)]],
                                    lp_v.at[pl.ds(g * GW, GW)], sem_lp))
    for cp in cps:
        cp.wait()
    cps = []
    for g in range(RPW // GW):
        cps.append(pltpu.async_copy(nch.at[lp_v.at[pl.ds(g * GW, GW)]],
                                    ncv_v.at[pl.ds(g * GW, GW)], sem_lp))
    val_cp = [pltpu.async_copy(valh.at[lp_v.at[pl.ds(0, CH)]],
                               vrows[0], sem_v[0])]
    for cp in cps:
        cp.wait()
    for cp in early:
        cp.wait()

    for k in range(RPW // L):
        oc = oc_v[pl.ds(k * L, L)]
        nc = ncv_v[pl.ds(k * L, L)]
        m_v[pl.ds(k * L, L)] = EMA * (oc / (oc + nc + 1e-8))

    out_cp = [None, None]
    for ch in range(NCH):
        b = ch & 1
        nb = (ch + 1) & 1
        rbase = base + ch * CH
        mem_cp[0].wait()
        val_cp[0].wait()
        if ch + 1 < NCH:
            mem_cp[0] = pltpu.async_copy(
                memh.at[idx_v.at[pl.ds(rbase + CH, CH)]], mrows[nb],
                sem_m[nb])
            val_cp[0] = pltpu.async_copy(
                valh.at[lp_v.at[pl.ds((ch + 1) * CH, CH)]], vrows[nb],
                sem_v[nb])
        if out_cp[b] is not None:
            out_cp[b].wait()

        mr = mrows[b]
        vr = vrows[b]
        orw = orows[b]

        def row_step(r, carry):
            olds = []
            vals = []
            acc = [jnp.zeros((L,), jnp.float32) for _ in range(6)]
            for k in range(NK):
                ok = mr[r, pl.ds(k * L, L)]
                vk = vr[r, pl.ds(k * L, L)]
                olds.append(ok)
                vals.append(vk)
                p = 3 * (k & 1)
                acc[p] = acc[p] + ok * ok
                acc[p + 1] = acc[p + 1] + vk * vk
                acc[p + 2] = acc[p + 2] + ok * vk
            so_f = jnp.full((L,), jnp.sum(acc[0] + acc[3]), jnp.float32)
            sv_f = jnp.full((L,), jnp.sum(acc[1] + acc[4]), jnp.float32)
            sov_f = jnp.full((L,), jnp.sum(acc[2] + acc[5]), jnp.float32)
            mv = jnp.full((L,), m_v[pl.ds(ch * CH + r, L)][0], jnp.float32)
            av = jnp.full((L,), aa_v[pl.ds(ch * CH + r, L)][0], jnp.float32)
            svv = _vrsqrt(sv_f)
            onsv = (1.0 - mv) * svv
            su_f = mv * mv * so_f + 2.0 * (mv * onsv) * sov_f \
                + onsv * onsv * sv_f
            suv = _vrsqrt(su_f)
            asu = av * suv
            c1 = asu * mv
            c2 = asu * onsv + (1.0 - av) * svv
            for k in range(NK):
                orw[r, pl.ds(k * L, L)] = c1 * olds[k] + c2 * vals[k]
            return carry

        lax.fori_loop(0, CH, row_step, 0)
        out_cp[b] = pltpu.async_copy(orw, outh.at[pl.ds(rbase, CH)],
                                     sem_o[b])
    for cp in out_cp:
        if cp is not None:
            cp.wait()


@jax.jit
def _run(mem, conf, act_f, idx, val, new_conf):
    mesh = plsc.VectorSubcoreMesh(core_axis_name="c", subcore_axis_name="s")
    k = functools.partial(
        pl.kernel, mesh=mesh,
        compiler_params=pltpu.CompilerParams(needs_layout_passes=False),
        out_type=(jax.ShapeDtypeStruct((B, D), jnp.float32),
                  jax.ShapeDtypeStruct((NC * PSZ,), jnp.int32)),
        scratch_types=[
            pltpu.VMEM((B,), jnp.int32),
            pltpu.VMEM((OWN,), jnp.int32),
            pltpu.VMEM((RPW,), jnp.int32),
            pltpu.VMEM((RPW,), jnp.int32),
            pltpu.VMEM((RPW,), jnp.float32),
            pltpu.VMEM((RPW + L,), jnp.float32),
            pltpu.VMEM((RPW,), jnp.float32),
            pltpu.VMEM((RPW + L,), jnp.float32),
            pltpu.VMEM((CH, D), jnp.float32),
            pltpu.VMEM((CH, D), jnp.float32),
            pltpu.VMEM((CH, D), jnp.float32),
            pltpu.VMEM((CH, D), jnp.float32),
            pltpu.VMEM((CH, D), jnp.float32),
            pltpu.VMEM((CH, D), jnp.float32),
            pltpu.SemaphoreType.DMA,
            pltpu.SemaphoreType.DMA,
            pltpu.SemaphoreType.DMA,
            pltpu.SemaphoreType.DMA,
            pltpu.SemaphoreType.DMA,
            pltpu.SemaphoreType.DMA,
            pltpu.SemaphoreType.DMA,
            pltpu.SemaphoreType.DMA,
        ])(_body)
    return k(mem, conf, act_f, idx, val, new_conf)


def kernel(mem, conf, is_active, idx, val, new_conf):
    out, _ = _run(mem, conf, is_active.astype(jnp.float32), idx, val,
                  new_conf)
    return out


# speedup vs baseline: 2.1313x; 2.1313x over previous
import functools

import jax
import jax.numpy as jnp
from jax import lax
from jax.experimental import pallas as pl
from jax.experimental.pallas import tpu as pltpu
from jax.experimental.pallas import tpu_sc as plsc

M, D, B = 100000, 256, 16384
EMA = 0.999
L = 16
NK = D // L
NC, NS = 2, 16
NW = NC * NS
RPW = B // NW
CH = 64
NCH = RPW // CH
OWN = 6256
PSZ = NS * OWN
NV = B // L
GW = 128

def _vrsqrt(s):
    s = jnp.maximum(s, 1e-24)
    i = plsc.bitcast(s, jnp.int32)
    y = plsc.bitcast(jnp.int32(0x5F3759DF) - (i >> 1), jnp.float32)
    for _ in range(3):
        y = y * (1.5 - 0.5 * s * y * y)
    return y


def _body(memh, confh, acth, idxh, valh, nch, outh, posh,
          idx_v, pos_v, gidx_v, lp_v, oc_v, aa_v, ncv_v, m_v,
          mrows0, mrows1, vrows0, vrows1, orows0, orows1,
          sem_s, sem_lp, sem_m0, sem_m1, sem_v0, sem_v1, sem_o0, sem_o1):
    c = lax.axis_index("c")
    s = lax.axis_index("s")
    wid = s * NC + c
    lane = lax.iota(jnp.int32, L)

    mrows = (mrows0, mrows1)
    vrows = (vrows0, vrows1)
    orows = (orows0, orows1)
    sem_m = (sem_m0, sem_m1)
    sem_v = (sem_v0, sem_v1)
    sem_o = (sem_o0, sem_o1)

    pltpu.sync_copy(idxh, idx_v)

    base = wid * RPW
    coff = c * PSZ

    early = []
    for g in range(RPW // GW):
        early.append(pltpu.async_copy(
            confh.at[idx_v.at[pl.ds(base + g * GW, GW)]],
            oc_v.at[pl.ds(g * GW, GW)], sem_s))
        early.append(pltpu.async_copy(
            acth.at[idx_v.at[pl.ds(base + g * GW, GW)]],
            aa_v.at[pl.ds(g * GW, GW)], sem_s))
    mem_cp = [pltpu.async_copy(memh.at[idx_v.at[pl.ds(base, CH)]],
                               mrows[0], sem_m[0])]

    for k in range(RPW // L):
        gidx_v[pl.ds(k * L, L)] = idx_v[pl.ds(base + k * L, L)] + coff

    lo = s * OWN

    shift_idx = jnp.minimum(lane + 1, L - 1)
    lastlane = lane == L - 1
    shift_dnums = lax.GatherDimensionNumbers(
        offset_dims=(), collapsed_slice_dims=(0,), start_index_map=(0,))

    def pos_step(v, carry):
        iv = idx_v[pl.ds(v * L, L)]
        key = iv * L + lane
        sk, sj = plsc.sort_key_val(key, v * L + lane)
        skid = sk >> 4
        nxt = lax.gather(skid, shift_idx[:, None], shift_dnums,
                         slice_sizes=(1,),
                         mode=lax.GatherScatterMode.PROMISE_IN_BOUNDS)
        is_end = jnp.logical_or(skid != nxt, lastlane)
        loc = skid - lo
        own = plsc.bitcast(loc, jnp.uint32) < jnp.uint32(OWN)
        mask = jnp.logical_and(is_end, own)
        plsc.store_scatter(pos_v, [loc], sj, mask=mask)
        return carry

    lax.fori_loop(0, NV, pos_step, 0, unroll=4)

    pltpu.sync_copy(pos_v, posh.at[pl.ds(c * PSZ + lo, OWN)])
    plsc.subcore_barrier()

    cps = []
    for g in range(RPW // GW):
        cps.append(pltpu.async_copy(posh.at[gidx_v.at[pl.ds(g * GW, GW)]],
                                    lp_v.at[pl.ds(g * GW, GW)], sem_lp))
    for cp in cps:
        cp.wait()
    cps = []
    for g in range(RPW // GW):
        cps.append(pltpu.async_copy(nch.at[lp_v.at[pl.ds(g * GW, GW)]],
                                    ncv_v.at[pl.ds(g * GW, GW)], sem_lp))
    val_cp = [pltpu.async_copy(valh.at[lp_v.at[pl.ds(0, CH)]],
                               vrows[0], sem_v[0])]
    for cp in cps:
        cp.wait()
    for cp in early:
        cp.wait()

    for k in range(RPW // L):
        oc = oc_v[pl.ds(k * L, L)]
        nc = ncv_v[pl.ds(k * L, L)]
        m_v[pl.ds(k * L, L)] = EMA * (oc / (oc + nc + 1e-8))

    out_cp = [None, None]
    for ch in range(NCH):
        b = ch & 1
        nb = (ch + 1) & 1
        rbase = base + ch * CH
        mem_cp[0].wait()
        val_cp[0].wait()
        if ch + 1 < NCH:
            mem_cp[0] = pltpu.async_copy(
                memh.at[idx_v.at[pl.ds(rbase + CH, CH)]], mrows[nb],
                sem_m[nb])
            val_cp[0] = pltpu.async_copy(
                valh.at[lp_v.at[pl.ds((ch + 1) * CH, CH)]], vrows[nb],
                sem_v[nb])
        if out_cp[b] is not None:
            out_cp[b].wait()

        mr = mrows[b]
        vr = vrows[b]
        orw = orows[b]

        def row_step(r, carry):
            olds = []
            vals = []
            acc = [jnp.zeros((L,), jnp.float32) for _ in range(6)]
            for k in range(NK):
                ok = mr[r, pl.ds(k * L, L)]
                vk = vr[r, pl.ds(k * L, L)]
                olds.append(ok)
                vals.append(vk)
                p = 3 * (k & 1)
                acc[p] = acc[p] + ok * ok
                acc[p + 1] = acc[p + 1] + vk * vk
                acc[p + 2] = acc[p + 2] + ok * vk
            so_f = jnp.full((L,), jnp.sum(acc[0] + acc[3]), jnp.float32)
            sv_f = jnp.full((L,), jnp.sum(acc[1] + acc[4]), jnp.float32)
            sov_f = jnp.full((L,), jnp.sum(acc[2] + acc[5]), jnp.float32)
            mv = jnp.full((L,), m_v[pl.ds(ch * CH + r, L)][0], jnp.float32)
            av = jnp.full((L,), aa_v[pl.ds(ch * CH + r, L)][0], jnp.float32)
            svv = _vrsqrt(sv_f)
            onsv = (1.0 - mv) * svv
            su_f = mv * mv * so_f + 2.0 * (mv * onsv) * sov_f \
                + onsv * onsv * sv_f
            suv = _vrsqrt(su_f)
            asu = av * suv
            c1 = asu * mv
            c2 = asu * onsv + (1.0 - av) * svv
            for k in range(NK):
                orw[r, pl.ds(k * L, L)] = c1 * olds[k] + c2 * vals[k]
            return carry

        lax.fori_loop(0, CH, row_step, 0)
        out_cp[b] = pltpu.async_copy(orw, outh.at[pl.ds(rbase, CH)],
                                     sem_o[b])
    for cp in out_cp:
        if cp is not None:
            cp.wait()


@jax.jit
def _run(mem, conf, act_f, idx, val, new_conf):
    mesh = plsc.VectorSubcoreMesh(core_axis_name="c", subcore_axis_name="s")
    k = functools.partial(
        pl.kernel, mesh=mesh,
        compiler_params=pltpu.CompilerParams(needs_layout_passes=False),
        out_type=(jax.ShapeDtypeStruct((B, D), jnp.float32),
                  jax.ShapeDtypeStruct((NC * PSZ,), jnp.int32)),
        scratch_types=[
            pltpu.VMEM((B,), jnp.int32),
            pltpu.VMEM((OWN,), jnp.int32),
            pltpu.VMEM((RPW,), jnp.int32),
            pltpu.VMEM((RPW,), jnp.int32),
            pltpu.VMEM((RPW,), jnp.float32),
            pltpu.VMEM((RPW + L,), jnp.float32),
            pltpu.VMEM((RPW,), jnp.float32),
            pltpu.VMEM((RPW + L,), jnp.float32),
            pltpu.VMEM((CH, D), jnp.float32),
            pltpu.VMEM((CH, D), jnp.float32),
            pltpu.VMEM((CH, D), jnp.float32),
            pltpu.VMEM((CH, D), jnp.float32),
            pltpu.VMEM((CH, D), jnp.float32),
            pltpu.VMEM((CH, D), jnp.float32),
            pltpu.SemaphoreType.DMA,
            pltpu.SemaphoreType.DMA,
            pltpu.SemaphoreType.DMA,
            pltpu.SemaphoreType.DMA,
            pltpu.SemaphoreType.DMA,
            pltpu.SemaphoreType.DMA,
            pltpu.SemaphoreType.DMA,
            pltpu.SemaphoreType.DMA,
        ])(_body)
    return k(mem, conf, act_f, idx, val, new_conf)


def kernel(mem, conf, is_active, idx, val, new_conf):
    out, _ = _run(mem, conf, is_active.astype(jnp.float32), idx, val,
                  new_conf)
    return out


# speedup vs baseline: 2.2692x; 1.0647x over previous
import functools

import jax
import jax.numpy as jnp
from jax import lax
from jax.experimental import pallas as pl
from jax.experimental.pallas import tpu as pltpu
from jax.experimental.pallas import tpu_sc as plsc

M, D, B = 100000, 256, 16384
EMA = 0.999
L = 16
NK = D // L
NC, NS = 2, 16
NW = NC * NS
RPW = B // NW
CH = 64
NCH = RPW // CH
OWN = 6256
PSZ = NS * OWN
NV = B // L
GW = 128

_SHIFT_DNUMS = lax.GatherDimensionNumbers(
    offset_dims=(), collapsed_slice_dims=(0,), start_index_map=(0,))


def _vrsqrt(s):
    s = jnp.maximum(s, 1e-24)
    i = plsc.bitcast(s, jnp.int32)
    y = plsc.bitcast(jnp.int32(0x5F3759DF) - (i >> 1), jnp.float32)
    for _ in range(3):
        y = y * (1.5 - 0.5 * s * y * y)
    return y


def _body(memh, confh, acth, idxh, valh, nch, outh, posh,
          idx_v, pos_v, gidx_v, lp_v, oc_v, aa_v, ncv_v, m_v,
          mrows0, mrows1, vrows0, vrows1, orows0, orows1,
          sem_s, sem_lp, sem_m0, sem_m1, sem_v0, sem_v1, sem_o0, sem_o1):
    c = lax.axis_index("c")
    s = lax.axis_index("s")
    wid = s * NC + c
    lane = lax.iota(jnp.int32, L)
    shift_idx = jnp.minimum(lane + 1, L - 1)
    lastlane = lane == L - 1

    mrows = (mrows0, mrows1)
    vrows = (vrows0, vrows1)
    orows = (orows0, orows1)
    sem_m = (sem_m0, sem_m1)
    sem_v = (sem_v0, sem_v1)
    sem_o = (sem_o0, sem_o1)

    pltpu.sync_copy(idxh, idx_v)

    base = wid * RPW
    coff = c * PSZ

    early = []
    for g in range(RPW // GW):
        early.append(pltpu.async_copy(
            confh.at[idx_v.at[pl.ds(base + g * GW, GW)]],
            oc_v.at[pl.ds(g * GW, GW)], sem_s))
        early.append(pltpu.async_copy(
            acth.at[idx_v.at[pl.ds(base + g * GW, GW)]],
            aa_v.at[pl.ds(g * GW, GW)], sem_s))
    mem_cp = [pltpu.async_copy(memh.at[idx_v.at[pl.ds(base, CH)]],
                               mrows[0], sem_m[0])]

    for k in range(RPW // L):
        gidx_v[pl.ds(k * L, L)] = idx_v[pl.ds(base + k * L, L)] + coff

    lo = s * OWN

    def pos_step(v, carry):
        iv = idx_v[pl.ds(v * L, L)]
        key = iv * L + lane
        sk, sj = plsc.sort_key_val(key, v * L + lane)
        skid = sk >> 4
        nxt = lax.gather(skid, shift_idx[:, None], _SHIFT_DNUMS,
                         slice_sizes=(1,),
                         mode=lax.GatherScatterMode.PROMISE_IN_BOUNDS)
        is_end = jnp.logical_or(skid != nxt, lastlane)
        loc = skid - lo
        own = plsc.bitcast(loc, jnp.uint32) < jnp.uint32(OWN)
        mask = jnp.logical_and(is_end, own)
        plsc.store_scatter(pos_v, [loc], sj, mask=mask)
        return carry

    lax.fori_loop(0, NV, pos_step, 0, unroll=4)

    pltpu.sync_copy(pos_v, posh.at[pl.ds(c * PSZ + lo, OWN)])
    plsc.subcore_barrier()

    cps = []
    for g in range(RPW // GW):
        cps.append(pltpu.async_copy(posh.at[gidx_v.at[pl.ds(g * GW, GW)]],
                                    lp_v.at[pl.ds(g * GW, GW)], sem_lp))
    for cp in cps:
        cp.wait()
    cps = []
    for g in range(RPW // GW):
        cps.append(pltpu.async_copy(nch.at[lp_v.at[pl.ds(g * GW, GW)]],
                                    ncv_v.at[pl.ds(g * GW, GW)], sem_lp))
    val_cp = [pltpu.async_copy(valh.at[lp_v.at[pl.ds(0, CH)]],
                               vrows[0], sem_v[0])]
    for cp in cps:
        cp.wait()
    for cp in early:
        cp.wait()

    for k in range(RPW // L):
        oc = oc_v[pl.ds(k * L, L)]
        nc = ncv_v[pl.ds(k * L, L)]
        m_v[pl.ds(k * L, L)] = EMA * (oc / (oc + nc + 1e-8))

    out_cp = [None, None]
    for ch in range(NCH):
        b = ch & 1
        nb = (ch + 1) & 1
        rbase = base + ch * CH
        mem_cp[0].wait()
        val_cp[0].wait()
        if ch + 1 < NCH:
            mem_cp[0] = pltpu.async_copy(
                memh.at[idx_v.at[pl.ds(rbase + CH, CH)]], mrows[nb],
                sem_m[nb])
            val_cp[0] = pltpu.async_copy(
                valh.at[lp_v.at[pl.ds((ch + 1) * CH, CH)]], vrows[nb],
                sem_v[nb])
        if out_cp[b] is not None:
            out_cp[b].wait()

        mr = mrows[b]
        vr = vrows[b]
        orw = orows[b]

        def row_step(r, carry):
            acc = [jnp.zeros((L,), jnp.float32) for _ in range(6)]
            for k in range(NK):
                ok = mr[r, pl.ds(k * L, L)]
                vk = vr[r, pl.ds(k * L, L)]
                p = 3 * (k & 1)
                acc[p] = acc[p] + ok * ok
                acc[p + 1] = acc[p + 1] + vk * vk
                acc[p + 2] = acc[p + 2] + ok * vk
            so_f = jnp.full((L,), jnp.sum(acc[0] + acc[3]), jnp.float32)
            sv_f = jnp.full((L,), jnp.sum(acc[1] + acc[4]), jnp.float32)
            sov_f = jnp.full((L,), jnp.sum(acc[2] + acc[5]), jnp.float32)
            mv = jnp.full((L,), m_v[pl.ds(ch * CH + r, L)][0], jnp.float32)
            av = jnp.full((L,), aa_v[pl.ds(ch * CH + r, L)][0], jnp.float32)
            svv = _vrsqrt(sv_f)
            onsv = (1.0 - mv) * svv
            su_f = mv * mv * so_f + 2.0 * (mv * onsv) * sov_f \
                + onsv * onsv * sv_f
            suv = _vrsqrt(su_f)
            asu = av * suv
            c1 = asu * mv
            c2 = asu * onsv + (1.0 - av) * svv
            for k in range(NK):
                orw[r, pl.ds(k * L, L)] = (c1 * mr[r, pl.ds(k * L, L)]
                                           + c2 * vr[r, pl.ds(k * L, L)])
            return carry

        lax.fori_loop(0, CH, row_step, 0, unroll=2)
        out_cp[b] = pltpu.async_copy(orw, outh.at[pl.ds(rbase, CH)],
                                     sem_o[b])
    for cp in out_cp:
        if cp is not None:
            cp.wait()


@jax.jit
def _run(mem, conf, act_f, idx, val, new_conf):
    mesh = plsc.VectorSubcoreMesh(core_axis_name="c", subcore_axis_name="s")
    k = functools.partial(
        pl.kernel, mesh=mesh,
        compiler_params=pltpu.CompilerParams(needs_layout_passes=False),
        out_type=(jax.ShapeDtypeStruct((B, D), jnp.float32),
                  jax.ShapeDtypeStruct((NC * PSZ,), jnp.int32)),
        scratch_types=[
            pltpu.VMEM((B,), jnp.int32),
            pltpu.VMEM((OWN,), jnp.int32),
            pltpu.VMEM((RPW,), jnp.int32),
            pltpu.VMEM((RPW,), jnp.int32),
            pltpu.VMEM((RPW,), jnp.float32),
            pltpu.VMEM((RPW + L,), jnp.float32),
            pltpu.VMEM((RPW,), jnp.float32),
            pltpu.VMEM((RPW + L,), jnp.float32),
            pltpu.VMEM((CH, D), jnp.float32),
            pltpu.VMEM((CH, D), jnp.float32),
            pltpu.VMEM((CH, D), jnp.float32),
            pltpu.VMEM((CH, D), jnp.float32),
            pltpu.VMEM((CH, D), jnp.float32),
            pltpu.VMEM((CH, D), jnp.float32),
            pltpu.SemaphoreType.DMA,
            pltpu.SemaphoreType.DMA,
            pltpu.SemaphoreType.DMA,
            pltpu.SemaphoreType.DMA,
            pltpu.SemaphoreType.DMA,
            pltpu.SemaphoreType.DMA,
            pltpu.SemaphoreType.DMA,
            pltpu.SemaphoreType.DMA,
        ])(_body)
    return k(mem, conf, act_f, idx, val, new_conf)


def kernel(mem, conf, is_active, idx, val, new_conf):
    out, _ = _run(mem, conf, is_active.astype(jnp.float32), idx, val,
                  new_conf)
    return out


# speedup vs baseline: 2.2783x; 1.0040x over previous
import functools

import jax
import jax.numpy as jnp
from jax import lax
from jax.experimental import pallas as pl
from jax.experimental.pallas import tpu as pltpu
from jax.experimental.pallas import tpu_sc as plsc

M, D, B = 100000, 256, 16384
EMA = 0.999
L = 16
NK = D // L
NC, NS = 2, 16
NW = NC * NS
RPW = B // NW
CH = 64
NCH = RPW // CH
OWN = 6256
PSZ = NS * OWN
NV = B // L
GW = 128

_SHIFT_DNUMS = lax.GatherDimensionNumbers(
    offset_dims=(), collapsed_slice_dims=(0,), start_index_map=(0,))


def _vrsqrt(s):
    s = jnp.maximum(s, 1e-24)
    i = plsc.bitcast(s, jnp.int32)
    y = plsc.bitcast(jnp.int32(0x5F3759DF) - (i >> 1), jnp.float32)
    for _ in range(3):
        y = y * (1.5 - 0.5 * s * y * y)
    return y


def _body(memh, confh, acth, idxh, valh, nch, outh, posh,
          idx_v, pos_v, gidx_v, lp_v, oc_v, aa_v, ncv_v, m_v,
          mrows0, mrows1, vrows0, vrows1, orows0, orows1,
          sem_s, sem_lp, sem_m0, sem_m1, sem_v0, sem_v1, sem_o0, sem_o1):
    c = lax.axis_index("c")
    s = lax.axis_index("s")
    wid = s * NC + c
    lane = lax.iota(jnp.int32, L)
    shift_idx = jnp.minimum(lane + 1, L - 1)
    lastlane = lane == L - 1

    mrows = (mrows0, mrows1)
    vrows = (vrows0, vrows1)
    orows = (orows0, orows1)
    sem_m = (sem_m0, sem_m1)
    sem_v = (sem_v0, sem_v1)
    sem_o = (sem_o0, sem_o1)

    pltpu.sync_copy(idxh, idx_v)

    base = wid * RPW
    coff = c * PSZ

    early = []
    for g in range(RPW // GW):
        early.append(pltpu.async_copy(
            confh.at[idx_v.at[pl.ds(base + g * GW, GW)]],
            oc_v.at[pl.ds(g * GW, GW)], sem_s))
        early.append(pltpu.async_copy(
            acth.at[idx_v.at[pl.ds(base + g * GW, GW)]],
            aa_v.at[pl.ds(g * GW, GW)], sem_s))
    mem_cp = [pltpu.async_copy(memh.at[idx_v.at[pl.ds(base, CH)]],
                               mrows[0], sem_m[0])]

    for k in range(RPW // L):
        gidx_v[pl.ds(k * L, L)] = idx_v[pl.ds(base + k * L, L)] + coff

    lo = s * OWN

    rot_idx = [jnp.minimum(lane + s_, L - 1) for s_ in range(1, L)]

    def pos_step(v, carry):
        iv = idx_v[pl.ds(v * L, L)]
        dup = None
        for ri in rot_idx:
            g2 = lax.gather(iv, ri[:, None], _SHIFT_DNUMS,
                            slice_sizes=(1,),
                            mode=lax.GatherScatterMode.PROMISE_IN_BOUNDS)
            eq = iv == g2
            dup = eq if dup is None else jnp.logical_or(dup, eq)
        keep = jnp.logical_or(jnp.logical_not(dup), lastlane)
        loc = iv - lo
        own = plsc.bitcast(loc, jnp.uint32) < jnp.uint32(OWN)
        mask = jnp.logical_and(keep, own)
        plsc.store_scatter(pos_v, [loc], v * L + lane, mask=mask)
        return carry

    lax.fori_loop(0, NV, pos_step, 0, unroll=2)

    pltpu.sync_copy(pos_v, posh.at[pl.ds(c * PSZ + lo, OWN)])
    plsc.subcore_barrier()

    cps = []
    for g in range(RPW // GW):
        cps.append(pltpu.async_copy(posh.at[gidx_v.at[pl.ds(g * GW, GW)]],
                                    lp_v.at[pl.ds(g * GW, GW)], sem_lp))
    for cp in cps:
        cp.wait()
    cps = []
    for g in range(RPW // GW):
        cps.append(pltpu.async_copy(nch.at[lp_v.at[pl.ds(g * GW, GW)]],
                                    ncv_v.at[pl.ds(g * GW, GW)], sem_lp))
    val_cp = [pltpu.async_copy(valh.at[lp_v.at[pl.ds(0, CH)]],
                               vrows[0], sem_v[0])]
    for cp in cps:
        cp.wait()
    for cp in early:
        cp.wait()

    for k in range(RPW // L):
        oc = oc_v[pl.ds(k * L, L)]
        nc = ncv_v[pl.ds(k * L, L)]
        m_v[pl.ds(k * L, L)] = EMA * (oc / (oc + nc + 1e-8))

    out_cp = [None, None]
    for ch in range(NCH):
        b = ch & 1
        nb = (ch + 1) & 1
        rbase = base + ch * CH
        mem_cp[0].wait()
        val_cp[0].wait()
        if ch + 1 < NCH:
            mem_cp[0] = pltpu.async_copy(
                memh.at[idx_v.at[pl.ds(rbase + CH, CH)]], mrows[nb],
                sem_m[nb])
            val_cp[0] = pltpu.async_copy(
                valh.at[lp_v.at[pl.ds((ch + 1) * CH, CH)]], vrows[nb],
                sem_v[nb])
        if out_cp[b] is not None:
            out_cp[b].wait()

        mr = mrows[b]
        vr = vrows[b]
        orw = orows[b]

        def row_step(r, carry):
            acc = [jnp.zeros((L,), jnp.float32) for _ in range(6)]
            for k in range(NK):
                ok = mr[r, pl.ds(k * L, L)]
                vk = vr[r, pl.ds(k * L, L)]
                p = 3 * (k & 1)
                acc[p] = acc[p] + ok * ok
                acc[p + 1] = acc[p + 1] + vk * vk
                acc[p + 2] = acc[p + 2] + ok * vk
            so_f = jnp.full((L,), jnp.sum(acc[0] + acc[3]), jnp.float32)
            sv_f = jnp.full((L,), jnp.sum(acc[1] + acc[4]), jnp.float32)
            sov_f = jnp.full((L,), jnp.sum(acc[2] + acc[5]), jnp.float32)
            mv = jnp.full((L,), m_v[pl.ds(ch * CH + r, L)][0], jnp.float32)
            av = jnp.full((L,), aa_v[pl.ds(ch * CH + r, L)][0], jnp.float32)
            svv = _vrsqrt(sv_f)
            onsv = (1.0 - mv) * svv
            su_f = mv * mv * so_f + 2.0 * (mv * onsv) * sov_f \
                + onsv * onsv * sv_f
            suv = _vrsqrt(su_f)
            asu = av * suv
            c1 = asu * mv
            c2 = asu * onsv + (1.0 - av) * svv
            for k in range(NK):
                orw[r, pl.ds(k * L, L)] = (c1 * mr[r, pl.ds(k * L, L)]
                                           + c2 * vr[r, pl.ds(k * L, L)])
            return carry

        lax.fori_loop(0, CH, row_step, 0, unroll=2)
        out_cp[b] = pltpu.async_copy(orw, outh.at[pl.ds(rbase, CH)],
                                     sem_o[b])
    for cp in out_cp:
        if cp is not None:
            cp.wait()


@jax.jit
def _run(mem, conf, act_f, idx, val, new_conf):
    mesh = plsc.VectorSubcoreMesh(core_axis_name="c", subcore_axis_name="s")
    k = functools.partial(
        pl.kernel, mesh=mesh,
        compiler_params=pltpu.CompilerParams(needs_layout_passes=False),
        out_type=(jax.ShapeDtypeStruct((B, D), jnp.float32),
                  jax.ShapeDtypeStruct((NC * PSZ,), jnp.int32)),
        scratch_types=[
            pltpu.VMEM((B,), jnp.int32),
            pltpu.VMEM((OWN,), jnp.int32),
            pltpu.VMEM((RPW,), jnp.int32),
            pltpu.VMEM((RPW,), jnp.int32),
            pltpu.VMEM((RPW,), jnp.float32),
            pltpu.VMEM((RPW + L,), jnp.float32),
            pltpu.VMEM((RPW,), jnp.float32),
            pltpu.VMEM((RPW + L,), jnp.float32),
            pltpu.VMEM((CH, D), jnp.float32),
            pltpu.VMEM((CH, D), jnp.float32),
            pltpu.VMEM((CH, D), jnp.float32),
            pltpu.VMEM((CH, D), jnp.float32),
            pltpu.VMEM((CH, D), jnp.float32),
            pltpu.VMEM((CH, D), jnp.float32),
            pltpu.SemaphoreType.DMA,
            pltpu.SemaphoreType.DMA,
            pltpu.SemaphoreType.DMA,
            pltpu.SemaphoreType.DMA,
            pltpu.SemaphoreType.DMA,
            pltpu.SemaphoreType.DMA,
            pltpu.SemaphoreType.DMA,
            pltpu.SemaphoreType.DMA,
        ])(_body)
    return k(mem, conf, act_f, idx, val, new_conf)


def kernel(mem, conf, is_active, idx, val, new_conf):
    out, _ = _run(mem, conf, is_active.astype(jnp.float32), idx, val,
                  new_conf)
    return out


# speedup vs baseline: 2.2978x; 1.0086x over previous
import functools

import jax
import jax.numpy as jnp
from jax import lax
from jax.experimental import pallas as pl
from jax.experimental.pallas import tpu as pltpu
from jax.experimental.pallas import tpu_sc as plsc

M, D, B = 100000, 256, 16384
EMA = 0.999
L = 16
NK = D // L
NC, NS = 2, 16
NW = NC * NS
RPW = B // NW
CH = 64
NCH = RPW // CH
OWN = 6256
PSZ = NS * OWN
NV = B // L
GW = 128

_SHIFT_DNUMS = lax.GatherDimensionNumbers(
    offset_dims=(), collapsed_slice_dims=(0,), start_index_map=(0,))


def _vrsqrt(s):
    s = jnp.maximum(s, 1e-24)
    i = plsc.bitcast(s, jnp.int32)
    y = plsc.bitcast(jnp.int32(0x5F3759DF) - (i >> 1), jnp.float32)
    for _ in range(3):
        y = y * (1.5 - 0.5 * s * y * y)
    return y


def _body(memh, confh, acth, idxh, valh, nch, outh, posh,
          idx_v, pos_v, gidx_v, lp_v, oc_v, aa_v, ncv_v, m_v,
          mrows0, mrows1, vrows0, vrows1, orows0, orows1,
          sem_s, sem_lp, sem_m0, sem_m1, sem_v0, sem_v1, sem_o0, sem_o1):
    c = lax.axis_index("c")
    s = lax.axis_index("s")
    wid = s * NC + c
    lane = lax.iota(jnp.int32, L)
    shift_idx = jnp.minimum(lane + 1, L - 1)
    lastlane = lane == L - 1

    mrows = (mrows0, mrows1)
    vrows = (vrows0, vrows1)
    orows = (orows0, orows1)
    sem_m = (sem_m0, sem_m1)
    sem_v = (sem_v0, sem_v1)
    sem_o = (sem_o0, sem_o1)

    pltpu.sync_copy(idxh, idx_v)

    base = wid * RPW
    coff = c * PSZ

    early = []
    for g in range(RPW // GW):
        early.append(pltpu.async_copy(
            confh.at[idx_v.at[pl.ds(base + g * GW, GW)]],
            oc_v.at[pl.ds(g * GW, GW)], sem_s))
        early.append(pltpu.async_copy(
            acth.at[idx_v.at[pl.ds(base + g * GW, GW)]],
            aa_v.at[pl.ds(g * GW, GW)], sem_s))
    mem_cp = [pltpu.async_copy(memh.at[idx_v.at[pl.ds(base, CH)]],
                               mrows[0], sem_m[0])]

    for k in range(RPW // L):
        gidx_v[pl.ds(k * L, L)] = idx_v[pl.ds(base + k * L, L)] + coff

    lo = s * OWN

    rot_idx = [jnp.minimum(lane + s_, L - 1) for s_ in range(1, L)]

    def pos_step(v, carry):
        iv = idx_v[pl.ds(v * L, L)]
        dup = None
        for ri in rot_idx:
            g2 = lax.gather(iv, ri[:, None], _SHIFT_DNUMS,
                            slice_sizes=(1,),
                            mode=lax.GatherScatterMode.PROMISE_IN_BOUNDS)
            eq = iv == g2
            dup = eq if dup is None else jnp.logical_or(dup, eq)
        keep = jnp.logical_or(jnp.logical_not(dup), lastlane)
        loc = iv - lo
        own = plsc.bitcast(loc, jnp.uint32) < jnp.uint32(OWN)
        mask = jnp.logical_and(keep, own)
        plsc.store_scatter(pos_v, [loc], v * L + lane, mask=mask)
        return carry

    lax.fori_loop(0, NV, pos_step, 0, unroll=4)

    pltpu.sync_copy(pos_v, posh.at[pl.ds(c * PSZ + lo, OWN)])
    plsc.subcore_barrier()

    cps = []
    for g in range(RPW // GW):
        cps.append(pltpu.async_copy(posh.at[gidx_v.at[pl.ds(g * GW, GW)]],
                                    lp_v.at[pl.ds(g * GW, GW)], sem_lp))
    for cp in cps:
        cp.wait()
    cps = []
    for g in range(RPW // GW):
        cps.append(pltpu.async_copy(nch.at[lp_v.at[pl.ds(g * GW, GW)]],
                                    ncv_v.at[pl.ds(g * GW, GW)], sem_lp))
    val_cp = [pltpu.async_copy(valh.at[lp_v.at[pl.ds(0, CH)]],
                               vrows[0], sem_v[0])]
    for cp in cps:
        cp.wait()
    for cp in early:
        cp.wait()

    for k in range(RPW // L):
        oc = oc_v[pl.ds(k * L, L)]
        nc = ncv_v[pl.ds(k * L, L)]
        m_v[pl.ds(k * L, L)] = EMA * (oc / (oc + nc + 1e-8))

    out_cp = [None, None]
    for ch in range(NCH):
        b = ch & 1
        nb = (ch + 1) & 1
        rbase = base + ch * CH
        mem_cp[0].wait()
        val_cp[0].wait()
        if ch + 1 < NCH:
            mem_cp[0] = pltpu.async_copy(
                memh.at[idx_v.at[pl.ds(rbase + CH, CH)]], mrows[nb],
                sem_m[nb])
            val_cp[0] = pltpu.async_copy(
                valh.at[lp_v.at[pl.ds((ch + 1) * CH, CH)]], vrows[nb],
                sem_v[nb])
        if out_cp[b] is not None:
            out_cp[b].wait()

        mr = mrows[b]
        vr = vrows[b]
        orw = orows[b]

        def row_step(r, carry):
            acc = [jnp.zeros((L,), jnp.float32) for _ in range(6)]
            for k in range(NK):
                ok = mr[r, pl.ds(k * L, L)]
                vk = vr[r, pl.ds(k * L, L)]
                p = 3 * (k & 1)
                acc[p] = acc[p] + ok * ok
                acc[p + 1] = acc[p + 1] + vk * vk
                acc[p + 2] = acc[p + 2] + ok * vk
            so_f = jnp.full((L,), jnp.sum(acc[0] + acc[3]), jnp.float32)
            sv_f = jnp.full((L,), jnp.sum(acc[1] + acc[4]), jnp.float32)
            sov_f = jnp.full((L,), jnp.sum(acc[2] + acc[5]), jnp.float32)
            mv = jnp.full((L,), m_v[pl.ds(ch * CH + r, L)][0], jnp.float32)
            av = jnp.full((L,), aa_v[pl.ds(ch * CH + r, L)][0], jnp.float32)
            svv = _vrsqrt(sv_f)
            onsv = (1.0 - mv) * svv
            su_f = mv * mv * so_f + 2.0 * (mv * onsv) * sov_f \
                + onsv * onsv * sv_f
            suv = _vrsqrt(su_f)
            asu = av * suv
            c1 = asu * mv
            c2 = asu * onsv + (1.0 - av) * svv
            for k in range(NK):
                orw[r, pl.ds(k * L, L)] = (c1 * mr[r, pl.ds(k * L, L)]
                                           + c2 * vr[r, pl.ds(k * L, L)])
            return carry

        lax.fori_loop(0, CH, row_step, 0, unroll=2)
        out_cp[b] = pltpu.async_copy(orw, outh.at[pl.ds(rbase, CH)],
                                     sem_o[b])
    for cp in out_cp:
        if cp is not None:
            cp.wait()


@jax.jit
def _run(mem, conf, act_f, idx, val, new_conf):
    mesh = plsc.VectorSubcoreMesh(core_axis_name="c", subcore_axis_name="s")
    k = functools.partial(
        pl.kernel, mesh=mesh,
        compiler_params=pltpu.CompilerParams(needs_layout_passes=False),
        out_type=(jax.ShapeDtypeStruct((B, D), jnp.float32),
                  jax.ShapeDtypeStruct((NC * PSZ,), jnp.int32)),
        scratch_types=[
            pltpu.VMEM((B,), jnp.int32),
            pltpu.VMEM((OWN,), jnp.int32),
            pltpu.VMEM((RPW,), jnp.int32),
            pltpu.VMEM((RPW,), jnp.int32),
            pltpu.VMEM((RPW,), jnp.float32),
            pltpu.VMEM((RPW + L,), jnp.float32),
            pltpu.VMEM((RPW,), jnp.float32),
            pltpu.VMEM((RPW + L,), jnp.float32),
            pltpu.VMEM((CH, D), jnp.float32),
            pltpu.VMEM((CH, D), jnp.float32),
            pltpu.VMEM((CH, D), jnp.float32),
            pltpu.VMEM((CH, D), jnp.float32),
            pltpu.VMEM((CH, D), jnp.float32),
            pltpu.VMEM((CH, D), jnp.float32),
            pltpu.SemaphoreType.DMA,
            pltpu.SemaphoreType.DMA,
            pltpu.SemaphoreType.DMA,
            pltpu.SemaphoreType.DMA,
            pltpu.SemaphoreType.DMA,
            pltpu.SemaphoreType.DMA,
            pltpu.SemaphoreType.DMA,
            pltpu.SemaphoreType.DMA,
        ])(_body)
    return k(mem, conf, act_f, idx, val, new_conf)


def kernel(mem, conf, is_active, idx, val, new_conf):
    out, _ = _run(mem, conf, is_active.astype(jnp.float32), idx, val,
                  new_conf)
    return out


# speedup vs baseline: 2.3021x; 1.0019x over previous
import functools

import jax
import jax.numpy as jnp
from jax import lax
from jax.experimental import pallas as pl
from jax.experimental.pallas import tpu as pltpu
from jax.experimental.pallas import tpu_sc as plsc

M, D, B = 100000, 256, 16384
EMA = 0.999
L = 16
NK = D // L
NC, NS = 2, 16
NW = NC * NS
RPW = B // NW
CH = 64
NCH = RPW // CH
OWN = 6256
PSZ = NS * OWN
NV = B // L
GW = 128

_SHIFT_DNUMS = lax.GatherDimensionNumbers(
    offset_dims=(), collapsed_slice_dims=(0,), start_index_map=(0,))


def _vrsqrt(s):
    s = jnp.maximum(s, 1e-24)
    i = plsc.bitcast(s, jnp.int32)
    y = plsc.bitcast(jnp.int32(0x5F3759DF) - (i >> 1), jnp.float32)
    for _ in range(3):
        y = y * (1.5 - 0.5 * s * y * y)
    return y


def _body(memh, confh, acth, idxh, valh, nch, outh, posh,
          idx_v, pos_v, gidx_v, lp_v, oc_v, aa_v, ncv_v, m_v,
          mrows0, mrows1, vrows0, vrows1, orows0, orows1,
          sem_s, sem_lp, sem_m0, sem_m1, sem_v0, sem_v1, sem_o0, sem_o1):
    c = lax.axis_index("c")
    s = lax.axis_index("s")
    wid = s * NC + c
    lane = lax.iota(jnp.int32, L)
    lastlane = lane == L - 1

    mrows = (mrows0, mrows1)
    vrows = (vrows0, vrows1)
    orows = (orows0, orows1)
    sem_m = (sem_m0, sem_m1)
    sem_v = (sem_v0, sem_v1)
    sem_o = (sem_o0, sem_o1)

    pltpu.sync_copy(idxh, idx_v)

    base = wid * RPW
    coff = c * PSZ

    early = []
    for g in range(RPW // GW):
        early.append(pltpu.async_copy(
            confh.at[idx_v.at[pl.ds(base + g * GW, GW)]],
            oc_v.at[pl.ds(g * GW, GW)], sem_s))
        early.append(pltpu.async_copy(
            acth.at[idx_v.at[pl.ds(base + g * GW, GW)]],
            aa_v.at[pl.ds(g * GW, GW)], sem_s))
    mem_cp = [pltpu.async_copy(memh.at[idx_v.at[pl.ds(base, CH)]],
                               mrows[0], sem_m[0])]

    for k in range(RPW // L):
        gidx_v[pl.ds(k * L, L)] = idx_v[pl.ds(base + k * L, L)] + coff

    lo = s * OWN

    rot_idx = [jnp.minimum(lane + s_, L - 1) for s_ in range(1, L)]

    def pos_step(v, carry):
        iv = idx_v[pl.ds(v * L, L)]
        dup = None
        for ri in rot_idx:
            g2 = lax.gather(iv, ri[:, None], _SHIFT_DNUMS,
                            slice_sizes=(1,),
                            mode=lax.GatherScatterMode.PROMISE_IN_BOUNDS)
            eq = iv == g2
            dup = eq if dup is None else jnp.logical_or(dup, eq)
        keep = jnp.logical_or(jnp.logical_not(dup), lastlane)
        loc = iv - lo
        own = plsc.bitcast(loc, jnp.uint32) < jnp.uint32(OWN)
        mask = jnp.logical_and(keep, own)
        plsc.store_scatter(pos_v, [loc], v * L + lane, mask=mask)
        return carry

    lax.fori_loop(0, NV, pos_step, 0, unroll=4)

    pltpu.sync_copy(pos_v, posh.at[pl.ds(c * PSZ + lo, OWN)])
    plsc.subcore_barrier()

    cps = []
    for g in range(RPW // GW):
        cps.append(pltpu.async_copy(posh.at[gidx_v.at[pl.ds(g * GW, GW)]],
                                    lp_v.at[pl.ds(g * GW, GW)], sem_lp))
    for cp in cps:
        cp.wait()
    cps = []
    for g in range(RPW // GW):
        cps.append(pltpu.async_copy(nch.at[lp_v.at[pl.ds(g * GW, GW)]],
                                    ncv_v.at[pl.ds(g * GW, GW)], sem_lp))
    val_cp = [pltpu.async_copy(valh.at[lp_v.at[pl.ds(0, CH)]],
                               vrows[0], sem_v[0])]
    for cp in cps:
        cp.wait()
    for cp in early:
        cp.wait()

    for k in range(RPW // L):
        oc = oc_v[pl.ds(k * L, L)]
        nc = ncv_v[pl.ds(k * L, L)]
        m_v[pl.ds(k * L, L)] = EMA * (oc / (oc + nc + 1e-8))

    out_cp = [None, None]
    for ch in range(NCH):
        b = ch & 1
        nb = (ch + 1) & 1
        rbase = base + ch * CH
        mem_cp[0].wait()
        val_cp[0].wait()
        if ch + 1 < NCH:
            mem_cp[0] = pltpu.async_copy(
                memh.at[idx_v.at[pl.ds(rbase + CH, CH)]], mrows[nb],
                sem_m[nb])
            val_cp[0] = pltpu.async_copy(
                valh.at[lp_v.at[pl.ds((ch + 1) * CH, CH)]], vrows[nb],
                sem_v[nb])
        if out_cp[b] is not None:
            out_cp[b].wait()

        mr = mrows[b]
        vr = vrows[b]
        orw = orows[b]

        def row_step(r, carry):
            acc = [jnp.zeros((L,), jnp.float32) for _ in range(6)]
            for k in range(NK):
                ok = mr[r, pl.ds(k * L, L)]
                vk = vr[r, pl.ds(k * L, L)]
                p = 3 * (k & 1)
                acc[p] = acc[p] + ok * ok
                acc[p + 1] = acc[p + 1] + vk * vk
                acc[p + 2] = acc[p + 2] + ok * vk
            so_f = jnp.full((L,), jnp.sum(acc[0] + acc[3]), jnp.float32)
            sv_f = jnp.full((L,), jnp.sum(acc[1] + acc[4]), jnp.float32)
            sov_f = jnp.full((L,), jnp.sum(acc[2] + acc[5]), jnp.float32)
            mv = jnp.full((L,), m_v[pl.ds(ch * CH + r, L)][0], jnp.float32)
            av = jnp.full((L,), aa_v[pl.ds(ch * CH + r, L)][0], jnp.float32)
            svv = _vrsqrt(sv_f)
            onsv = (1.0 - mv) * svv
            su_f = mv * mv * so_f + 2.0 * (mv * onsv) * sov_f \
                + onsv * onsv * sv_f
            suv = _vrsqrt(su_f)
            asu = av * suv
            c1 = asu * mv
            c2 = asu * onsv + (1.0 - av) * svv
            for k in range(NK):
                orw[r, pl.ds(k * L, L)] = (c1 * mr[r, pl.ds(k * L, L)]
                                           + c2 * vr[r, pl.ds(k * L, L)])
            return carry

        lax.fori_loop(0, CH, row_step, 0, unroll=2)
        out_cp[b] = pltpu.async_copy(orw, outh.at[pl.ds(rbase, CH)],
                                     sem_o[b])
    for cp in out_cp:
        if cp is not None:
            cp.wait()


@jax.jit
def _run(mem, conf, act_f, idx, val, new_conf):
    mesh = plsc.VectorSubcoreMesh(core_axis_name="c", subcore_axis_name="s")
    k = functools.partial(
        pl.kernel, mesh=mesh,
        compiler_params=pltpu.CompilerParams(needs_layout_passes=False),
        out_type=(jax.ShapeDtypeStruct((B, D), jnp.float32),
                  jax.ShapeDtypeStruct((NC * PSZ,), jnp.int32)),
        scratch_types=[
            pltpu.VMEM((B,), jnp.int32),
            pltpu.VMEM((OWN,), jnp.int32),
            pltpu.VMEM((RPW,), jnp.int32),
            pltpu.VMEM((RPW,), jnp.int32),
            pltpu.VMEM((RPW,), jnp.float32),
            pltpu.VMEM((RPW + L,), jnp.float32),
            pltpu.VMEM((RPW,), jnp.float32),
            pltpu.VMEM((RPW + L,), jnp.float32),
            pltpu.VMEM((CH, D), jnp.float32),
            pltpu.VMEM((CH, D), jnp.float32),
            pltpu.VMEM((CH, D), jnp.float32),
            pltpu.VMEM((CH, D), jnp.float32),
            pltpu.VMEM((CH, D), jnp.float32),
            pltpu.VMEM((CH, D), jnp.float32),
            pltpu.SemaphoreType.DMA,
            pltpu.SemaphoreType.DMA,
            pltpu.SemaphoreType.DMA,
            pltpu.SemaphoreType.DMA,
            pltpu.SemaphoreType.DMA,
            pltpu.SemaphoreType.DMA,
            pltpu.SemaphoreType.DMA,
            pltpu.SemaphoreType.DMA,
        ])(_body)
    return k(mem, conf, act_f, idx, val, new_conf)


def kernel(mem, conf, is_active, idx, val, new_conf):
    out, _ = _run(mem, conf, is_active.astype(jnp.float32), idx, val,
                  new_conf)
    return out
